# Initial kernel scaffold; baseline (speedup 1.0000x reference)
import jax
import jax.numpy as jnp
from jax.experimental import pallas as pl


def kernel(x, edge_index, W, att_src, att_dst, bias, gamma, beta):
    raise NotImplementedError("write your pallas kernel here")


# speedup vs baseline: 30.8611x; 30.8611x over previous
import dataclasses

import jax
import jax.numpy as jnp
from jax import lax
from jax.experimental import pallas as pl
from jax.experimental.pallas import tpu as pltpu
from jax.experimental.pallas import tpu_sc as plsc

N_NODES = 10000
DIN = 128
NH = 8
NCH = 16
D = NH * NCH
LANES = 16

NPAD = 10240
STRIPE = NPAD // 16

NUM_EDGES = 320000 + N_NODES
B = 128
NWORK = 32
CHUNKS_C = -(-NUM_EDGES // (NWORK * B))
EPAD = CHUNKS_C * NWORK * B
SPAN_C = CHUNKS_C * B
CHUNKS_B = EPAD // (16 * B)
SPAN_B = CHUNKS_B * B


def _proj_body(x_ref, w_ref, a2s_ref, a2d_ref, h_ref, as_ref, ad_ref):
    h = jnp.dot(x_ref[...], w_ref[...], preferred_element_type=jnp.float32)
    h_ref[...] = h
    as_ref[...] = jnp.dot(h, a2s_ref[...], preferred_element_type=jnp.float32)
    ad_ref[...] = jnp.dot(h, a2d_ref[...], preferred_element_type=jnp.float32)


def _project(x_pad, W, a2s, a2d):
    blk = 512
    grid = (NPAD // blk,)
    return pl.pallas_call(
        _proj_body,
        grid=grid,
        in_specs=[
            pl.BlockSpec((blk, DIN), lambda i: (i, 0)),
            pl.BlockSpec((DIN, D), lambda i: (0, 0)),
            pl.BlockSpec((DIN, LANES), lambda i: (0, 0)),
            pl.BlockSpec((DIN, LANES), lambda i: (0, 0)),
        ],
        out_specs=[
            pl.BlockSpec((blk, D), lambda i: (i, 0)),
            pl.BlockSpec((blk, LANES), lambda i: (i, 0)),
            pl.BlockSpec((blk, LANES), lambda i: (i, 0)),
        ],
        out_shape=[
            jax.ShapeDtypeStruct((NPAD, D), jnp.float32),
            jax.ShapeDtypeStruct((NPAD, LANES), jnp.float32),
            jax.ShapeDtypeStruct((NPAD, LANES), jnp.float32),
        ],
    )(x_pad, W, a2s, a2d)


def _edge_body(src_h, dst_h, h_hbm, asrc_h, adst_h, out_h,
               out_spm, dst_spm,
               srcv, dstv, arows, brows, trow, hrows, sem):
    c = lax.axis_index("core")
    t = lax.axis_index("subcore")
    wid = c * 16 + t
    st = t * STRIPE
    lane = lax.iota(jnp.int32, 16)
    flip = lane ^ 8

    @pl.loop(0, B)
    def _(i):
        for j in range(NH):
            hrows[i, pl.ds(j * NCH, NCH)] = jnp.zeros((LANES,), jnp.float32)

    @pl.loop(0, STRIPE // B)
    def _(i):
        pltpu.sync_copy(hrows, out_spm.at[pl.ds(st + i * B, B)])

    pltpu.sync_copy(adst_h.at[pl.ds(st, STRIPE)],
                    dst_spm.at[pl.ds(st, STRIPE)])
    plsc.subcore_barrier()

    base_b = t * SPAN_B

    @pl.loop(0, CHUNKS_B)
    def _(g):
        base = base_b + g * B
        pltpu.sync_copy(src_h.at[pl.ds(base, B)], srcv)
        pltpu.sync_copy(dst_h.at[pl.ds(base, B)], dstv)
        pltpu.sync_copy(asrc_h.at[srcv], arows)
        pltpu.sync_copy(adst_h.at[dstv], brows)

        @pl.loop(0, B)
        def _(i):
            a = arows[i, :] + brows[i, :]
            a = jnp.minimum(a, jnp.float32(30.0))
            a = jnp.where(a > 0.0, a, a * jnp.float32(0.2))
            trow[:] = jnp.exp(a)
            gs = plsc.load_gather(trow, [flip])
            arows[i, :] = jnp.where(lane >= 8, gs, jnp.float32(0.0))

        pltpu.sync_copy(arows, dst_spm.at[dstv], add=True)

    plsc.subcore_barrier()

    base_c = wid * SPAN_C

    @pl.loop(0, CHUNKS_C)
    def _(g):
        base = base_c + g * B
        pltpu.sync_copy(src_h.at[pl.ds(base, B)], srcv)
        pltpu.sync_copy(dst_h.at[pl.ds(base, B)], dstv)
        cp = pltpu.async_copy(h_hbm.at[srcv], hrows, sem)
        pltpu.sync_copy(asrc_h.at[srcv], arows)
        pltpu.sync_copy(dst_spm.at[dstv], brows)
        cp.wait()

        @pl.loop(0, B)
        def _(i):
            a = arows[i, :] + brows[i, :]
            a = jnp.minimum(a, jnp.float32(30.0))
            a = jnp.where(a > 0.0, a, a * jnp.float32(0.2))
            gv = jnp.exp(a)
            srow = plsc.load_gather(brows, [jnp.full((16,), i, jnp.int32),
                                            flip])
            al = gv / (srow + jnp.float32(1e-16))
            for j in range(NH):
                aj = al[j]
                hrows[i, pl.ds(j * NCH, NCH)] = (
                    hrows[i, pl.ds(j * NCH, NCH)] * aj)

        pltpu.sync_copy(hrows, out_spm.at[dstv], add=True)

    plsc.subcore_barrier()

    @pl.loop(0, STRIPE // B)
    def _(i):
        pltpu.sync_copy(out_spm.at[pl.ds(st + i * B, B)],
                        out_h.at[c, pl.ds(st + i * B, B)])


def _edge_phase(src, dst, h, asrc_t, adst_t):
    mesh = plsc.VectorSubcoreMesh(core_axis_name="core",
                                  subcore_axis_name="subcore")
    cp = pltpu.CompilerParams(use_tc_tiling_on_sc=False)
    if "needs_layout_passes" in pltpu.CompilerParams.__dataclass_fields__:
        cp = dataclasses.replace(cp, needs_layout_passes=False)
    f = pl.kernel(
        _edge_body,
        compiler_params=cp,
        out_type=jax.ShapeDtypeStruct((2, NPAD, D), jnp.float32),
        mesh=mesh,
        scratch_types=[
            pltpu.VMEM_SHARED((NPAD, D), jnp.float32),
            pltpu.VMEM_SHARED((NPAD, LANES), jnp.float32),
            pltpu.VMEM((B,), jnp.int32),
            pltpu.VMEM((B,), jnp.int32),
            pltpu.VMEM((B, LANES), jnp.float32),
            pltpu.VMEM((B, LANES), jnp.float32),
            pltpu.VMEM((LANES,), jnp.float32),
            pltpu.VMEM((B, D), jnp.float32),
            pltpu.SemaphoreType.DMA,
        ],
    )
    return f(src, dst, h, asrc_t, adst_t)


def _finish_body(p0_ref, p1_ref, bias_ref, gamma_ref, beta_ref, o_ref):
    o = p0_ref[...] + p1_ref[...] + bias_ref[...]
    mu = jnp.mean(o, axis=-1, keepdims=True)
    d = o - mu
    var = jnp.mean(d * d, axis=-1, keepdims=True)
    o = d * lax.rsqrt(var + jnp.float32(1e-5))
    o = o * gamma_ref[...] + beta_ref[...]
    o_ref[...] = jnp.where(o > 0.0, o, jnp.exp(o) - jnp.float32(1.0))


def _finish(p0, p1, bias, gamma, beta):
    blk = 1000
    grid = (N_NODES // blk,)
    return pl.pallas_call(
        _finish_body,
        grid=grid,
        in_specs=[
            pl.BlockSpec((blk, D), lambda i: (i, 0)),
            pl.BlockSpec((blk, D), lambda i: (i, 0)),
            pl.BlockSpec((1, D), lambda i: (0, 0)),
            pl.BlockSpec((1, D), lambda i: (0, 0)),
            pl.BlockSpec((1, D), lambda i: (0, 0)),
        ],
        out_specs=pl.BlockSpec((blk, D), lambda i: (i, 0)),
        out_shape=jax.ShapeDtypeStruct((N_NODES, D), jnp.float32),
    )(p0, p1, bias, gamma, beta)


@jax.jit
def kernel(x, edge_index, W, att_src, att_dst, bias, gamma, beta):
    n = x.shape[0]
    loop = jnp.arange(n, dtype=jnp.int32)
    ei = jnp.concatenate(
        [edge_index.astype(jnp.int32), jnp.stack([loop, loop])], axis=1)
    src = jnp.pad(ei[0], (0, EPAD - NUM_EDGES), constant_values=n)
    dst = jnp.pad(ei[1], (0, EPAD - NUM_EDGES), constant_values=n)

    x_pad = jnp.pad(x, ((0, NPAD - n), (0, 0)))

    att_s = att_src.reshape(D)
    att_d = att_dst.reshape(D)
    col = jnp.arange(LANES, dtype=jnp.int32)
    head_of_row = jnp.arange(D, dtype=jnp.int32)[:, None] // NCH
    sel_s = (head_of_row == col[None, :] % NH).astype(jnp.float32)
    sel_d = ((head_of_row == col[None, :]) & (col[None, :] < NH)
             ).astype(jnp.float32)
    a2s = att_s[:, None] * sel_s
    a2d = att_d[:, None] * sel_d

    h, asrc_t, adst_t = _project(x_pad, W, a2s, a2d)
    out_part = _edge_phase(src, dst, h, asrc_t, adst_t)
    out = _finish(out_part[0, :N_NODES], out_part[1, :N_NODES],
                  bias.reshape(1, D), gamma.reshape(1, D),
                  beta.reshape(1, D))
    return out


# speedup vs baseline: 63.1427x; 2.0460x over previous
import dataclasses

import jax
import jax.numpy as jnp
from jax import lax
from jax.experimental import pallas as pl
from jax.experimental.pallas import tpu as pltpu
from jax.experimental.pallas import tpu_sc as plsc

N_NODES = 10000
DIN = 128
NH = 8
NCH = 16
D = NH * NCH
LANES = 16

NPAD = 10240
STRIPE = NPAD // 16

NUM_EDGES = 320000 + N_NODES
B = 64
NWORK = 32
CHC = -(-NUM_EDGES // (NWORK * B))
EPAD = CHC * NWORK * B
SPAN_C = CHC * B
CHB = EPAD // (16 * B)
SPAN_B = CHB * B

NB = 6


def _proj_body(x_ref, w_ref, a2s_ref, a2d_ref, h_ref, as_ref, ad_ref):
    h = jnp.dot(x_ref[...], w_ref[...], preferred_element_type=jnp.float32)
    h_ref[...] = h
    as_ref[...] = jnp.dot(h, a2s_ref[...], preferred_element_type=jnp.float32)
    ad_ref[...] = jnp.dot(h, a2d_ref[...], preferred_element_type=jnp.float32)


def _project(x_pad, W, a2s, a2d):
    blk = 512
    grid = (NPAD // blk,)
    return pl.pallas_call(
        _proj_body,
        grid=grid,
        in_specs=[
            pl.BlockSpec((blk, DIN), lambda i: (i, 0)),
            pl.BlockSpec((DIN, D), lambda i: (0, 0)),
            pl.BlockSpec((DIN, LANES), lambda i: (0, 0)),
            pl.BlockSpec((DIN, LANES), lambda i: (0, 0)),
        ],
        out_specs=[
            pl.BlockSpec((blk, D), lambda i: (i, 0)),
            pl.BlockSpec((blk, LANES), lambda i: (i, 0)),
            pl.BlockSpec((blk, LANES), lambda i: (i, 0)),
        ],
        out_shape=[
            jax.ShapeDtypeStruct((NPAD, D), jnp.float32),
            jax.ShapeDtypeStruct((NPAD, LANES), jnp.float32),
            jax.ShapeDtypeStruct((NPAD, LANES), jnp.float32),
        ],
    )(x_pad, W, a2s, a2d)


def _edge_body(src_h, dst_h, h_hbm, asrc_h, adst_h, out_h, g_hbm,
               s_spm, out_spm,
               srcv, dstv, arows, brows, grows, hrows,
               si0, si1, si2, si3, si4, si5, sg0, sg1, so0, so1):
    sem_i = [si0, si1, si2, si3, si4, si5]
    sem_g = [sg0, sg1]
    sem_o = [so0, so1]
    c = lax.axis_index("core")
    t = lax.axis_index("subcore")
    wid = c * 16 + t
    st = t * STRIPE

    @pl.loop(0, B)
    def _(i):
        for j in range(NH):
            hrows[0][i, pl.ds(j * NCH, NCH)] = jnp.zeros((LANES,),
                                                         jnp.float32)
        grows[0][i, :] = jnp.zeros((LANES,), jnp.float32)

    @pl.loop(0, STRIPE // B)
    def _(i):
        pltpu.sync_copy(hrows[0], out_spm.at[pl.ds(st + i * B, B)])
        pltpu.sync_copy(grows[0], s_spm.at[pl.ds(st + i * B, B)])

    rem = STRIPE - (STRIPE // B) * B
    if rem:
        pltpu.sync_copy(hrows[0].at[pl.ds(0, rem)],
                        out_spm.at[pl.ds(st + (STRIPE // B) * B, rem)])
        pltpu.sync_copy(grows[0].at[pl.ds(0, rem)],
                        s_spm.at[pl.ds(st + (STRIPE // B) * B, rem)])
    plsc.subcore_barrier()

    def issue_idx(base):
        d1 = pltpu.async_copy(src_h.at[pl.ds(base, B)], srcv[0], sem_i[0])
        d2 = pltpu.async_copy(dst_h.at[pl.ds(base, B)], dstv[0], sem_i[0])
        return (d1, d2)

    base_b = t * SPAN_B

    @pl.loop(0, CHB, step=NB)
    def _(jb):
        base0 = base_b + jb * B
        idesc = []
        for r in range(NB):
            d1 = pltpu.async_copy(src_h.at[pl.ds(base0 + r * B, B)],
                                  srcv[r], sem_i[r])
            d2 = pltpu.async_copy(dst_h.at[pl.ds(base0 + r * B, B)],
                                  dstv[r], sem_i[r])
            idesc.append((d1, d2))

        def issue_g(r):
            d1 = pltpu.async_copy(asrc_h.at[srcv[r]], arows[r % 2],
                                  sem_g[r % 2])
            d2 = pltpu.async_copy(adst_h.at[dstv[r]], brows[r % 2],
                                  sem_g[r % 2])
            return (d1, d2)

        idesc[0][0].wait()
        idesc[0][1].wait()
        gdesc = {0: issue_g(0)}
        odesc = {}
        for r in range(NB):
            if r + 1 < NB:
                idesc[r + 1][0].wait()
                idesc[r + 1][1].wait()
                gdesc[r + 1] = issue_g(r + 1)
            gdesc[r][0].wait()
            gdesc[r][1].wait()
            if r - 2 >= 0:
                odesc[r - 2].wait()
            d = r % 2

            @pl.loop(0, B, step=2)
            def _(i):
                for k in range(2):
                    e = i + k
                    a = arows[d][e, :] + brows[d][e, :]
                    a = jnp.minimum(a, jnp.float32(30.0))
                    a = jnp.where(a > 0.0, a, a * jnp.float32(0.2))
                    grows[d][e, :] = jnp.exp(a)

            pltpu.sync_copy(grows[d], s_spm.at[dstv[r]], add=True)
            odesc[r] = pltpu.async_copy(
                grows[d], g_hbm.at[c, pl.ds(base0 + r * B, B)], sem_o[d])
        odesc[NB - 2].wait()
        odesc[NB - 1].wait()

    plsc.subcore_barrier()

    base_c = wid * SPAN_C

    @pl.loop(0, CHC, step=NB)
    def _(jb):
        base0 = base_c + jb * B
        idesc = []
        for r in range(NB):
            d1 = pltpu.async_copy(src_h.at[pl.ds(base0 + r * B, B)],
                                  srcv[r], sem_i[r])
            d2 = pltpu.async_copy(dst_h.at[pl.ds(base0 + r * B, B)],
                                  dstv[r], sem_i[r])
            idesc.append((d1, d2))

        def issue_g(r):
            d1 = pltpu.async_copy(h_hbm.at[srcv[r]], hrows[r % 2],
                                  sem_g[r % 2])
            d2 = pltpu.async_copy(g_hbm.at[c, pl.ds(base0 + r * B, B)],
                                  grows[r % 2], sem_g[r % 2])
            return (d1, d2)

        idesc[0][0].wait()
        idesc[0][1].wait()
        gdesc = {0: issue_g(0)}
        for r in range(NB):
            if r + 1 < NB:
                idesc[r + 1][0].wait()
                idesc[r + 1][1].wait()
                gdesc[r + 1] = issue_g(r + 1)
            gdesc[r][0].wait()
            gdesc[r][1].wait()
            d = r % 2
            pltpu.sync_copy(s_spm.at[dstv[r]], brows[d])

            @pl.loop(0, B, step=2)
            def _(i):
                for k in range(2):
                    e = i + k
                    al = (grows[d][e, :]
                          / (brows[d][e, :] + jnp.float32(1e-16)))
                    for j in range(NH):
                        hrows[d][e, pl.ds(j * NCH, NCH)] = (
                            hrows[d][e, pl.ds(j * NCH, NCH)] * al[j])

            pltpu.sync_copy(hrows[d], out_spm.at[dstv[r]], add=True)

    plsc.subcore_barrier()

    @pl.loop(0, STRIPE // B)
    def _(i):
        pltpu.sync_copy(out_spm.at[pl.ds(st + i * B, B)],
                        out_h.at[c, pl.ds(st + i * B, B)])

    if rem:
        pltpu.sync_copy(out_spm.at[pl.ds(st + (STRIPE // B) * B, rem)],
                        out_h.at[c, pl.ds(st + (STRIPE // B) * B, rem)])


def _edge_phase(src, dst, h, asrc_t, adst_t):
    mesh = plsc.VectorSubcoreMesh(core_axis_name="core",
                                  subcore_axis_name="subcore")
    cp = pltpu.CompilerParams(use_tc_tiling_on_sc=False)
    if "needs_layout_passes" in pltpu.CompilerParams.__dataclass_fields__:
        cp = dataclasses.replace(cp, needs_layout_passes=False)
    f = pl.kernel(
        _edge_body,
        compiler_params=cp,
        out_type=(
            jax.ShapeDtypeStruct((2, NPAD, D), jnp.float32),
            jax.ShapeDtypeStruct((2, EPAD, LANES), jnp.float32),
        ),
        mesh=mesh,
        scratch_types=[
            pltpu.VMEM_SHARED((NPAD, LANES), jnp.float32),
            pltpu.VMEM_SHARED((NPAD, D), jnp.float32),
            [pltpu.VMEM((B,), jnp.int32) for _ in range(NB)],
            [pltpu.VMEM((B,), jnp.int32) for _ in range(NB)],
            [pltpu.VMEM((B, LANES), jnp.float32) for _ in range(2)],
            [pltpu.VMEM((B, LANES), jnp.float32) for _ in range(2)],
            [pltpu.VMEM((B, LANES), jnp.float32) for _ in range(2)],
            [pltpu.VMEM((B, D), jnp.float32) for _ in range(2)],
            pltpu.SemaphoreType.DMA,
            pltpu.SemaphoreType.DMA,
            pltpu.SemaphoreType.DMA,
            pltpu.SemaphoreType.DMA,
            pltpu.SemaphoreType.DMA,
            pltpu.SemaphoreType.DMA,
            pltpu.SemaphoreType.DMA,
            pltpu.SemaphoreType.DMA,
            pltpu.SemaphoreType.DMA,
            pltpu.SemaphoreType.DMA,
        ],
    )
    return f(src, dst, h, asrc_t, adst_t)


def _finish_body(p0_ref, p1_ref, bias_ref, gamma_ref, beta_ref, o_ref):
    o = p0_ref[...] + p1_ref[...] + bias_ref[...]
    mu = jnp.mean(o, axis=-1, keepdims=True)
    d = o - mu
    var = jnp.mean(d * d, axis=-1, keepdims=True)
    o = d * lax.rsqrt(var + jnp.float32(1e-5))
    o = o * gamma_ref[...] + beta_ref[...]
    o_ref[...] = jnp.where(o > 0.0, o, jnp.exp(o) - jnp.float32(1.0))


def _finish(p0, p1, bias, gamma, beta):
    blk = 1000
    grid = (N_NODES // blk,)
    return pl.pallas_call(
        _finish_body,
        grid=grid,
        in_specs=[
            pl.BlockSpec((blk, D), lambda i: (i, 0)),
            pl.BlockSpec((blk, D), lambda i: (i, 0)),
            pl.BlockSpec((1, D), lambda i: (0, 0)),
            pl.BlockSpec((1, D), lambda i: (0, 0)),
            pl.BlockSpec((1, D), lambda i: (0, 0)),
        ],
        out_specs=pl.BlockSpec((blk, D), lambda i: (i, 0)),
        out_shape=jax.ShapeDtypeStruct((N_NODES, D), jnp.float32),
    )(p0, p1, bias, gamma, beta)


@jax.jit
def kernel(x, edge_index, W, att_src, att_dst, bias, gamma, beta):
    n = x.shape[0]
    loop = jnp.arange(n, dtype=jnp.int32)
    ei = jnp.concatenate(
        [edge_index.astype(jnp.int32), jnp.stack([loop, loop])], axis=1)
    src = jnp.pad(ei[0], (0, EPAD - NUM_EDGES), constant_values=n)
    dst = jnp.pad(ei[1], (0, EPAD - NUM_EDGES), constant_values=n)

    x_pad = jnp.pad(x, ((0, NPAD - n), (0, 0)))

    att_s = att_src.reshape(D)
    att_d = att_dst.reshape(D)
    col = jnp.arange(LANES, dtype=jnp.int32)
    head_of_row = jnp.arange(D, dtype=jnp.int32)[:, None] // NCH
    sel_s = (head_of_row == col[None, :] % NH).astype(jnp.float32)
    sel_d = ((head_of_row == col[None, :]) & (col[None, :] < NH)
             ).astype(jnp.float32)
    a2s = att_s[:, None] * sel_s
    a2d = att_d[:, None] * sel_d

    h, asrc_t, adst_t = _project(x_pad, W, a2s, a2d)
    out_part, _ = _edge_phase(src, dst, h, asrc_t, adst_t)
    out = _finish(out_part[0, :N_NODES], out_part[1, :N_NODES],
                  bias.reshape(1, D), gamma.reshape(1, D),
                  beta.reshape(1, D))
    return out


# speedup vs baseline: 66.6694x; 1.0559x over previous
import dataclasses

import jax
import jax.numpy as jnp
from jax import lax
from jax.experimental import pallas as pl
from jax.experimental.pallas import tpu as pltpu
from jax.experimental.pallas import tpu_sc as plsc

N_NODES = 10000
DIN = 128
NH = 8
NCH = 16
D = NH * NCH
LANES = 16

NPAD = 10240
STRIPE = NPAD // 16

NUM_EDGES = 320000 + N_NODES
B = 64
NWORK = 32
CHC = -(-NUM_EDGES // (NWORK * B))
EPAD = CHC * NWORK * B
SPAN_C = CHC * B
CHB = EPAD // (16 * B)
SPAN_B = CHB * B

NB = 9


def _proj_body(x_ref, w_ref, a2s_ref, a2d_ref, h_ref, as_ref, ad_ref):
    h = jnp.dot(x_ref[...], w_ref[...], preferred_element_type=jnp.float32)
    h_ref[...] = h
    as_ref[...] = jnp.dot(h, a2s_ref[...], preferred_element_type=jnp.float32)
    ad_ref[...] = jnp.dot(h, a2d_ref[...], preferred_element_type=jnp.float32)


def _project(x_pad, W, a2s, a2d):
    blk = 512
    grid = (NPAD // blk,)
    return pl.pallas_call(
        _proj_body,
        grid=grid,
        in_specs=[
            pl.BlockSpec((blk, DIN), lambda i: (i, 0)),
            pl.BlockSpec((DIN, D), lambda i: (0, 0)),
            pl.BlockSpec((DIN, LANES), lambda i: (0, 0)),
            pl.BlockSpec((DIN, LANES), lambda i: (0, 0)),
        ],
        out_specs=[
            pl.BlockSpec((blk, D), lambda i: (i, 0)),
            pl.BlockSpec((blk, LANES), lambda i: (i, 0)),
            pl.BlockSpec((blk, LANES), lambda i: (i, 0)),
        ],
        out_shape=[
            jax.ShapeDtypeStruct((NPAD, D), jnp.float32),
            jax.ShapeDtypeStruct((NPAD, LANES), jnp.float32),
            jax.ShapeDtypeStruct((NPAD, LANES), jnp.float32),
        ],
    )(x_pad, W, a2s, a2d)


def _edge_body(src_h, dst_h, h_hbm, asrc_h, adst_h, out_h, g_hbm,
               s_spm, out_spm,
               srcv, dstv, arows, brows, grows, hrows,
               si0, si1, si2, si3, si4, si5, si6, si7, si8,
               sg0, sg1, so0, so1, ss0, ss1):
    sem_i = [si0, si1, si2, si3, si4, si5, si6, si7, si8]
    sem_g = [sg0, sg1]
    sem_o = [so0, so1]
    sem_s = [ss0, ss1]
    c = lax.axis_index("core")
    t = lax.axis_index("subcore")
    wid = c * 16 + t
    st = t * STRIPE

    @pl.loop(0, B)
    def _(i):
        for j in range(NH):
            hrows[0][i, pl.ds(j * NCH, NCH)] = jnp.zeros((LANES,),
                                                         jnp.float32)
        grows[0][i, :] = jnp.zeros((LANES,), jnp.float32)

    @pl.loop(0, STRIPE // B)
    def _(i):
        pltpu.sync_copy(hrows[0], out_spm.at[pl.ds(st + i * B, B)])
        pltpu.sync_copy(grows[0], s_spm.at[pl.ds(st + i * B, B)])

    rem = STRIPE - (STRIPE // B) * B
    if rem:
        pltpu.sync_copy(hrows[0].at[pl.ds(0, rem)],
                        out_spm.at[pl.ds(st + (STRIPE // B) * B, rem)])
        pltpu.sync_copy(grows[0].at[pl.ds(0, rem)],
                        s_spm.at[pl.ds(st + (STRIPE // B) * B, rem)])
    plsc.subcore_barrier()

    def issue_idx(base):
        d1 = pltpu.async_copy(src_h.at[pl.ds(base, B)], srcv[0], sem_i[0])
        d2 = pltpu.async_copy(dst_h.at[pl.ds(base, B)], dstv[0], sem_i[0])
        return (d1, d2)

    base_b = t * SPAN_B

    @pl.loop(0, CHB, step=NB)
    def _(jb):
        base0 = base_b + jb * B
        idesc = []
        for r in range(NB):
            d1 = pltpu.async_copy(src_h.at[pl.ds(base0 + r * B, B)],
                                  srcv[r], sem_i[r])
            d2 = pltpu.async_copy(dst_h.at[pl.ds(base0 + r * B, B)],
                                  dstv[r], sem_i[r])
            idesc.append((d1, d2))

        def issue_g(r):
            d1 = pltpu.async_copy(asrc_h.at[srcv[r]], arows[r % 2],
                                  sem_g[r % 2])
            d2 = pltpu.async_copy(adst_h.at[dstv[r]], brows[r % 2],
                                  sem_g[r % 2])
            return (d1, d2)

        idesc[0][0].wait()
        idesc[0][1].wait()
        gdesc = {0: issue_g(0)}
        odesc = {}
        sdesc = {}
        for r in range(NB):
            if r + 1 < NB:
                idesc[r + 1][0].wait()
                idesc[r + 1][1].wait()
                gdesc[r + 1] = issue_g(r + 1)
            gdesc[r][0].wait()
            gdesc[r][1].wait()
            if r - 2 >= 0:
                odesc[r - 2].wait()
                sdesc[r - 2].wait()
            d = r % 2

            @pl.loop(0, B, step=2)
            def _(i):
                for k in range(2):
                    e = i + k
                    a = arows[d][e, :] + brows[d][e, :]
                    a = jnp.minimum(a, jnp.float32(30.0))
                    a = jnp.where(a > 0.0, a, a * jnp.float32(0.2))
                    grows[d][e, :] = jnp.exp(a)

            sdesc[r] = pltpu.async_copy(grows[d], s_spm.at[dstv[r]],
                                        sem_s[d], add=True)
            odesc[r] = pltpu.async_copy(
                grows[d], g_hbm.at[c, pl.ds(base0 + r * B, B)], sem_o[d])
        for r in (NB - 2, NB - 1):
            odesc[r].wait()
            sdesc[r].wait()

    plsc.subcore_barrier()

    base_c = wid * SPAN_C

    @pl.loop(0, CHC, step=NB)
    def _(jb):
        base0 = base_c + jb * B
        idesc = []
        for r in range(NB):
            d1 = pltpu.async_copy(src_h.at[pl.ds(base0 + r * B, B)],
                                  srcv[r], sem_i[r])
            d2 = pltpu.async_copy(dst_h.at[pl.ds(base0 + r * B, B)],
                                  dstv[r], sem_i[r])
            idesc.append((d1, d2))

        def issue_g(r):
            d1 = pltpu.async_copy(h_hbm.at[srcv[r]], hrows[r % 2],
                                  sem_g[r % 2])
            d2 = pltpu.async_copy(g_hbm.at[c, pl.ds(base0 + r * B, B)],
                                  grows[r % 2], sem_g[r % 2])
            return (d1, d2)

        idesc[0][0].wait()
        idesc[0][1].wait()
        gdesc = {0: issue_g(0)}
        sdesc = {}
        for r in range(NB):
            if r - 1 >= 0:
                sdesc[r - 1].wait()
            if r + 1 < NB:
                idesc[r + 1][0].wait()
                idesc[r + 1][1].wait()
                gdesc[r + 1] = issue_g(r + 1)
            gdesc[r][0].wait()
            gdesc[r][1].wait()
            d = r % 2
            pltpu.sync_copy(s_spm.at[dstv[r]], brows[d])

            @pl.loop(0, B, step=4)
            def _(i):
                for k in range(4):
                    e = i + k
                    al = (grows[d][e, :]
                          / (brows[d][e, :] + jnp.float32(1e-16)))
                    for j in range(NH):
                        hrows[d][e, pl.ds(j * NCH, NCH)] = (
                            hrows[d][e, pl.ds(j * NCH, NCH)] * al[j])

            sdesc[r] = pltpu.async_copy(hrows[d], out_spm.at[dstv[r]],
                                        sem_s[d], add=True)
        sdesc[NB - 1].wait()

    plsc.subcore_barrier()

    @pl.loop(0, STRIPE // B)
    def _(i):
        pltpu.sync_copy(out_spm.at[pl.ds(st + i * B, B)],
                        out_h.at[c, pl.ds(st + i * B, B)])

    if rem:
        pltpu.sync_copy(out_spm.at[pl.ds(st + (STRIPE // B) * B, rem)],
                        out_h.at[c, pl.ds(st + (STRIPE // B) * B, rem)])


def _edge_phase(src, dst, h, asrc_t, adst_t):
    mesh = plsc.VectorSubcoreMesh(core_axis_name="core",
                                  subcore_axis_name="subcore")
    cp = pltpu.CompilerParams(use_tc_tiling_on_sc=False)
    if "needs_layout_passes" in pltpu.CompilerParams.__dataclass_fields__:
        cp = dataclasses.replace(cp, needs_layout_passes=False)
    f = pl.kernel(
        _edge_body,
        compiler_params=cp,
        out_type=(
            jax.ShapeDtypeStruct((2, NPAD, D), jnp.float32),
            jax.ShapeDtypeStruct((2, EPAD, LANES), jnp.float32),
        ),
        mesh=mesh,
        scratch_types=[
            pltpu.VMEM_SHARED((NPAD, LANES), jnp.float32),
            pltpu.VMEM_SHARED((NPAD, D), jnp.float32),
            [pltpu.VMEM((B,), jnp.int32) for _ in range(NB)],
            [pltpu.VMEM((B,), jnp.int32) for _ in range(NB)],
            [pltpu.VMEM((B, LANES), jnp.float32) for _ in range(2)],
            [pltpu.VMEM((B, LANES), jnp.float32) for _ in range(2)],
            [pltpu.VMEM((B, LANES), jnp.float32) for _ in range(2)],
            [pltpu.VMEM((B, D), jnp.float32) for _ in range(2)],
        ] + [pltpu.SemaphoreType.DMA] * 15,
    )
    return f(src, dst, h, asrc_t, adst_t)


def _finish_body(p0_ref, p1_ref, bias_ref, gamma_ref, beta_ref, o_ref):
    o = p0_ref[...] + p1_ref[...] + bias_ref[...]
    mu = jnp.mean(o, axis=-1, keepdims=True)
    d = o - mu
    var = jnp.mean(d * d, axis=-1, keepdims=True)
    o = d * lax.rsqrt(var + jnp.float32(1e-5))
    o = o * gamma_ref[...] + beta_ref[...]
    o_ref[...] = jnp.where(o > 0.0, o, jnp.exp(o) - jnp.float32(1.0))


def _finish(p0, p1, bias, gamma, beta):
    blk = 1000
    grid = (N_NODES // blk,)
    return pl.pallas_call(
        _finish_body,
        grid=grid,
        in_specs=[
            pl.BlockSpec((blk, D), lambda i: (i, 0)),
            pl.BlockSpec((blk, D), lambda i: (i, 0)),
            pl.BlockSpec((1, D), lambda i: (0, 0)),
            pl.BlockSpec((1, D), lambda i: (0, 0)),
            pl.BlockSpec((1, D), lambda i: (0, 0)),
        ],
        out_specs=pl.BlockSpec((blk, D), lambda i: (i, 0)),
        out_shape=jax.ShapeDtypeStruct((N_NODES, D), jnp.float32),
    )(p0, p1, bias, gamma, beta)


@jax.jit
def kernel(x, edge_index, W, att_src, att_dst, bias, gamma, beta):
    n = x.shape[0]
    loop = jnp.arange(n, dtype=jnp.int32)
    ei = jnp.concatenate(
        [edge_index.astype(jnp.int32), jnp.stack([loop, loop])], axis=1)
    src = jnp.pad(ei[0], (0, EPAD - NUM_EDGES), constant_values=n)
    dst = jnp.pad(ei[1], (0, EPAD - NUM_EDGES), constant_values=n)

    x_pad = jnp.pad(x, ((0, NPAD - n), (0, 0)))

    att_s = att_src.reshape(D)
    att_d = att_dst.reshape(D)
    col = jnp.arange(LANES, dtype=jnp.int32)
    head_of_row = jnp.arange(D, dtype=jnp.int32)[:, None] // NCH
    sel_s = (head_of_row == col[None, :] % NH).astype(jnp.float32)
    sel_d = ((head_of_row == col[None, :]) & (col[None, :] < NH)
             ).astype(jnp.float32)
    a2s = att_s[:, None] * sel_s
    a2d = att_d[:, None] * sel_d

    h, asrc_t, adst_t = _project(x_pad, W, a2s, a2d)
    out_part, _ = _edge_phase(src, dst, h, asrc_t, adst_t)
    out = _finish(out_part[0, :N_NODES], out_part[1, :N_NODES],
                  bias.reshape(1, D), gamma.reshape(1, D),
                  beta.reshape(1, D))
    return out


# speedup vs baseline: 66.6739x; 1.0001x over previous
import dataclasses

import jax
import jax.numpy as jnp
from jax import lax
from jax.experimental import pallas as pl
from jax.experimental.pallas import tpu as pltpu
from jax.experimental.pallas import tpu_sc as plsc

N_NODES = 10000
DIN = 128
NH = 8
NCH = 16
D = NH * NCH
LANES = 16

NPAD = 10240
STRIPE = NPAD // 16

NUM_EDGES = 320000 + N_NODES
B = 64
NWORK = 32
CHC = -(-NUM_EDGES // (NWORK * B))
EPAD = CHC * NWORK * B
SPAN_C = CHC * B
CHB = EPAD // (16 * B)
SPAN_B = CHB * B

NB = 9


def _proj_body(x_ref, w_ref, a2s_ref, a2d_ref, h_ref, as_ref, ad_ref):
    h = jnp.dot(x_ref[...], w_ref[...], preferred_element_type=jnp.float32)
    h_ref[...] = h
    as_ref[...] = jnp.dot(h, a2s_ref[...], preferred_element_type=jnp.float32)
    ad_ref[...] = jnp.dot(h, a2d_ref[...], preferred_element_type=jnp.float32)


def _project(x_pad, W, a2s, a2d):
    blk = 512
    grid = (NPAD // blk,)
    return pl.pallas_call(
        _proj_body,
        grid=grid,
        in_specs=[
            pl.BlockSpec((blk, DIN), lambda i: (i, 0)),
            pl.BlockSpec((DIN, D), lambda i: (0, 0)),
            pl.BlockSpec((DIN, LANES), lambda i: (0, 0)),
            pl.BlockSpec((DIN, LANES), lambda i: (0, 0)),
        ],
        out_specs=[
            pl.BlockSpec((blk, D), lambda i: (i, 0)),
            pl.BlockSpec((blk, LANES), lambda i: (i, 0)),
            pl.BlockSpec((blk, LANES), lambda i: (i, 0)),
        ],
        out_shape=[
            jax.ShapeDtypeStruct((NPAD, D), jnp.float32),
            jax.ShapeDtypeStruct((NPAD, LANES), jnp.float32),
            jax.ShapeDtypeStruct((NPAD, LANES), jnp.float32),
        ],
    )(x_pad, W, a2s, a2d)


def _edge_body(src_h, dst_h, h_hbm, asrc_h, adst_h, out_h, g_hbm,
               s_spm, out_spm,
               srcv, dstv, arows, brows, grows, hrows,
               si0, si1, si2, si3, si4, si5, si6, si7, si8,
               sg0, sg1, so0, so1, ss0, ss1):
    sem_i = [si0, si1, si2, si3, si4, si5, si6, si7, si8]
    sem_g = [sg0, sg1]
    sem_o = [so0, so1]
    sem_s = [ss0, ss1]
    c = lax.axis_index("core")
    t = lax.axis_index("subcore")
    wid = c * 16 + t
    st = t * STRIPE

    @pl.loop(0, B)
    def _(i):
        for j in range(NH):
            hrows[0][i, pl.ds(j * NCH, NCH)] = jnp.zeros((LANES,),
                                                         jnp.float32)
        grows[0][i, :] = jnp.zeros((LANES,), jnp.float32)

    @pl.loop(0, STRIPE // B)
    def _(i):
        pltpu.sync_copy(hrows[0], out_spm.at[pl.ds(st + i * B, B)])
        pltpu.sync_copy(grows[0], s_spm.at[pl.ds(st + i * B, B)])

    rem = STRIPE - (STRIPE // B) * B
    if rem:
        pltpu.sync_copy(hrows[0].at[pl.ds(0, rem)],
                        out_spm.at[pl.ds(st + (STRIPE // B) * B, rem)])
        pltpu.sync_copy(grows[0].at[pl.ds(0, rem)],
                        s_spm.at[pl.ds(st + (STRIPE // B) * B, rem)])
    plsc.subcore_barrier()

    def issue_idx(base):
        d1 = pltpu.async_copy(src_h.at[pl.ds(base, B)], srcv[0], sem_i[0])
        d2 = pltpu.async_copy(dst_h.at[pl.ds(base, B)], dstv[0], sem_i[0])
        return (d1, d2)

    base_b = t * SPAN_B

    @pl.loop(0, CHB, step=NB)
    def _(jb):
        base0 = base_b + jb * B
        idesc = []
        for r in range(NB):
            d1 = pltpu.async_copy(src_h.at[pl.ds(base0 + r * B, B)],
                                  srcv[r], sem_i[r])
            d2 = pltpu.async_copy(dst_h.at[pl.ds(base0 + r * B, B)],
                                  dstv[r], sem_i[r])
            idesc.append((d1, d2))

        def issue_g(r):
            d1 = pltpu.async_copy(asrc_h.at[srcv[r]], arows[r % 2],
                                  sem_g[r % 2])
            d2 = pltpu.async_copy(adst_h.at[dstv[r]], brows[r % 2],
                                  sem_g[r % 2])
            return (d1, d2)

        idesc[0][0].wait()
        idesc[0][1].wait()
        gdesc = {0: issue_g(0)}
        odesc = {}
        sdesc = {}
        for r in range(NB):
            if r + 1 < NB:
                idesc[r + 1][0].wait()
                idesc[r + 1][1].wait()
                gdesc[r + 1] = issue_g(r + 1)
            gdesc[r][0].wait()
            gdesc[r][1].wait()
            if r - 2 >= 0:
                odesc[r - 2].wait()
                sdesc[r - 2].wait()
            d = r % 2

            @pl.loop(0, B, step=2)
            def _(i):
                for k in range(2):
                    e = i + k
                    a = arows[d][e, :] + brows[d][e, :]
                    a = jnp.minimum(a, jnp.float32(30.0))
                    a = jnp.where(a > 0.0, a, a * jnp.float32(0.2))
                    grows[d][e, :] = jnp.exp(a)

            sdesc[r] = pltpu.async_copy(grows[d], s_spm.at[dstv[r]],
                                        sem_s[d], add=True)
            odesc[r] = pltpu.async_copy(
                grows[d], g_hbm.at[c, pl.ds(base0 + r * B, B)], sem_o[d])
        for r in (NB - 2, NB - 1):
            odesc[r].wait()
            sdesc[r].wait()

    plsc.subcore_barrier()

    base_c = wid * SPAN_C

    @pl.loop(0, CHC, step=NB)
    def _(jb):
        base0 = base_c + jb * B
        idesc = []
        for r in range(NB):
            d1 = pltpu.async_copy(src_h.at[pl.ds(base0 + r * B, B)],
                                  srcv[r], sem_i[r])
            d2 = pltpu.async_copy(dst_h.at[pl.ds(base0 + r * B, B)],
                                  dstv[r], sem_i[r])
            idesc.append((d1, d2))

        def issue_g(r):
            d1 = pltpu.async_copy(h_hbm.at[srcv[r]], hrows[r % 2],
                                  sem_g[r % 2])
            d2 = pltpu.async_copy(g_hbm.at[c, pl.ds(base0 + r * B, B)],
                                  grows[r % 2], sem_g[r % 2])
            return (d1, d2)

        idesc[0][0].wait()
        idesc[0][1].wait()
        gdesc = {0: issue_g(0)}
        sdesc = {}
        for r in range(NB):
            if r - 1 >= 0:
                sdesc[r - 1].wait()
            if r + 1 < NB:
                idesc[r + 1][0].wait()
                idesc[r + 1][1].wait()
                gdesc[r + 1] = issue_g(r + 1)
            for g_ in gdesc[r]:
                g_.wait()
            d = r % 2
            pltpu.sync_copy(s_spm.at[dstv[r]], brows[d])

            @pl.loop(0, B, step=4)
            def _(i):
                for k in range(4):
                    e = i + k
                    al = (grows[d][e, :]
                          / (brows[d][e, :] + jnp.float32(1e-16)))
                    for j in range(NH):
                        hrows[d][e, pl.ds(j * NCH, NCH)] = (
                            hrows[d][e, pl.ds(j * NCH, NCH)] * al[j])

            sdesc[r] = pltpu.async_copy(hrows[d], out_spm.at[dstv[r]],
                                        sem_s[d], add=True)
        sdesc[NB - 1].wait()

    plsc.subcore_barrier()

    @pl.loop(0, STRIPE // B)
    def _(i):
        pltpu.sync_copy(out_spm.at[pl.ds(st + i * B, B)],
                        out_h.at[c, pl.ds(st + i * B, B)])

    if rem:
        pltpu.sync_copy(out_spm.at[pl.ds(st + (STRIPE // B) * B, rem)],
                        out_h.at[c, pl.ds(st + (STRIPE // B) * B, rem)])


def _edge_phase(src, dst, h, asrc_t, adst_t):
    mesh = plsc.VectorSubcoreMesh(core_axis_name="core",
                                  subcore_axis_name="subcore")
    cp = pltpu.CompilerParams(use_tc_tiling_on_sc=False)
    if "needs_layout_passes" in pltpu.CompilerParams.__dataclass_fields__:
        cp = dataclasses.replace(cp, needs_layout_passes=False)
    f = pl.kernel(
        _edge_body,
        compiler_params=cp,
        out_type=(
            jax.ShapeDtypeStruct((2, NPAD, D), jnp.float32),
            jax.ShapeDtypeStruct((2, EPAD, LANES), jnp.float32),
        ),
        mesh=mesh,
        scratch_types=[
            pltpu.VMEM_SHARED((NPAD, LANES), jnp.float32),
            pltpu.VMEM_SHARED((NPAD, D), jnp.float32),
            [pltpu.VMEM((B,), jnp.int32) for _ in range(NB)],
            [pltpu.VMEM((B,), jnp.int32) for _ in range(NB)],
            [pltpu.VMEM((B, LANES), jnp.float32) for _ in range(2)],
            [pltpu.VMEM((B, LANES), jnp.float32) for _ in range(2)],
            [pltpu.VMEM((B, LANES), jnp.float32) for _ in range(2)],
            [pltpu.VMEM((B, D), jnp.float32) for _ in range(2)],
        ] + [pltpu.SemaphoreType.DMA] * 15,
    )
    return f(src, dst, h, asrc_t, adst_t)


def _finish_body(p0_ref, p1_ref, bias_ref, gamma_ref, beta_ref, o_ref):
    o = p0_ref[...] + p1_ref[...] + bias_ref[...]
    mu = jnp.mean(o, axis=-1, keepdims=True)
    d = o - mu
    var = jnp.mean(d * d, axis=-1, keepdims=True)
    o = d * lax.rsqrt(var + jnp.float32(1e-5))
    o = o * gamma_ref[...] + beta_ref[...]
    o_ref[...] = jnp.where(o > 0.0, o, jnp.exp(o) - jnp.float32(1.0))


def _finish(p0, p1, bias, gamma, beta):
    blk = 1000
    grid = (N_NODES // blk,)
    return pl.pallas_call(
        _finish_body,
        grid=grid,
        in_specs=[
            pl.BlockSpec((blk, D), lambda i: (i, 0)),
            pl.BlockSpec((blk, D), lambda i: (i, 0)),
            pl.BlockSpec((1, D), lambda i: (0, 0)),
            pl.BlockSpec((1, D), lambda i: (0, 0)),
            pl.BlockSpec((1, D), lambda i: (0, 0)),
        ],
        out_specs=pl.BlockSpec((blk, D), lambda i: (i, 0)),
        out_shape=jax.ShapeDtypeStruct((N_NODES, D), jnp.float32),
    )(p0, p1, bias, gamma, beta)


@jax.jit
def kernel(x, edge_index, W, att_src, att_dst, bias, gamma, beta):
    n = x.shape[0]
    loop = jnp.arange(n, dtype=jnp.int32)
    ei = jnp.concatenate(
        [edge_index.astype(jnp.int32), jnp.stack([loop, loop])], axis=1)
    src = jnp.pad(ei[0], (0, EPAD - NUM_EDGES), constant_values=n)
    dst = jnp.pad(ei[1], (0, EPAD - NUM_EDGES), constant_values=n)

    x_pad = jnp.pad(x, ((0, NPAD - n), (0, 0)))

    att_s = att_src.reshape(D)
    att_d = att_dst.reshape(D)
    col = jnp.arange(LANES, dtype=jnp.int32)
    head_of_row = jnp.arange(D, dtype=jnp.int32)[:, None] // NCH
    sel_s = (head_of_row == col[None, :] % NH).astype(jnp.float32)
    sel_d = ((head_of_row == col[None, :]) & (col[None, :] < NH)
             ).astype(jnp.float32)
    a2s = att_s[:, None] * sel_s
    a2d = att_d[:, None] * sel_d

    h, asrc_t, adst_t = _project(x_pad, W, a2s, a2d)
    out_part, _ = _edge_phase(src, dst, h, asrc_t, adst_t)
    out = _finish(out_part[0, :N_NODES], out_part[1, :N_NODES],
                  bias.reshape(1, D), gamma.reshape(1, D),
                  beta.reshape(1, D))
    return out
